# baseline (speedup 1.0000x reference)
import jax
import jax.numpy as jnp
from jax import lax
from jax.experimental import pallas as pl
from jax.experimental.pallas import tpu as pltpu
from jax.experimental.pallas import tpu_sc as plsc

B = 16384
D = 16

_info = plsc.get_sparse_core_info()
NC, NS, L = _info.num_cores, _info.num_subcores, _info.num_lanes
NW = NC * NS
BW = B // NW
NCHUNK = BW // 128
NGRP = BW // L


def _body(u_hbm, i_hbm,
          ubm_hbm, ubl_hbm, uvm_hbm, uvl_hbm,
          ibm_hbm, ibl_hbm, ivm_hbm, ivl_hbm,
          glob_hbm, ebu_hbm, evu_hbm, ebi_hbm, evi_hbm,
          out_hbm,
          u2d, i2d,
          bmu_u, blv_u, bmu_i, blv_i,
          vmu_u, vlv_u, vmu_i, vlv_i,
          ebu_v, ebi_v, evu_v, evi_v,
          glob_v, out_v, sem):
    wid = lax.axis_index("s") * NC + lax.axis_index("c")
    base = wid * BW

    for k in range(NCHUNK):
        pltpu.sync_copy(u_hbm.at[pl.ds(base + k * 128, 128)], u2d.at[k])
        pltpu.sync_copy(i_hbm.at[pl.ds(base + k * 128, 128)], i2d.at[k])
    pltpu.sync_copy(glob_hbm, glob_v.at[pl.ds(0, 1)])

    copies = []
    for k in range(NCHUNK):
        sl = pl.ds(k * 128, 128)
        uk, ik = u2d.at[k], i2d.at[k]
        copies.append(pltpu.async_copy(uvm_hbm.at[uk], vmu_u.at[sl, :], sem))
        copies.append(pltpu.async_copy(uvl_hbm.at[uk], vlv_u.at[sl, :], sem))
        copies.append(pltpu.async_copy(ivm_hbm.at[ik], vmu_i.at[sl, :], sem))
        copies.append(pltpu.async_copy(ivl_hbm.at[ik], vlv_i.at[sl, :], sem))
        copies.append(pltpu.async_copy(ubm_hbm.at[uk], bmu_u.at[sl], sem))
        copies.append(pltpu.async_copy(ubl_hbm.at[uk], blv_u.at[sl], sem))
        copies.append(pltpu.async_copy(ibm_hbm.at[ik], bmu_i.at[sl], sem))
        copies.append(pltpu.async_copy(ibl_hbm.at[ik], blv_i.at[sl], sem))
    bsl = pl.ds(base, BW)
    copies.append(pltpu.async_copy(evu_hbm.at[bsl, :], evu_v, sem))
    copies.append(pltpu.async_copy(evi_hbm.at[bsl, :], evi_v, sem))
    copies.append(pltpu.async_copy(ebu_hbm.at[bsl], ebu_v, sem))
    copies.append(pltpu.async_copy(ebi_hbm.at[bsl], ebi_v, sem))
    for c in copies:
        c.wait()

    iota = lax.iota(jnp.int32, L)
    zz = jnp.zeros((L,), jnp.int32)
    glob_vec = lax.gather(
        glob_v[...], zz.reshape(L, 1),
        dimension_numbers=lax.GatherDimensionNumbers(
            offset_dims=(), collapsed_slice_dims=(0,), start_index_map=(0,)),
        slice_sizes=(1,), mode=lax.GatherScatterMode.PROMISE_IN_BOUNDS)

    def group(g, carry):
        rows = g * L + iota
        gsl = pl.ds(g * L, L)
        gb = bmu_u[gsl] + jnp.exp(0.5 * blv_u[gsl]) * ebu_v[gsl] \
            + bmu_i[gsl] + jnp.exp(0.5 * blv_i[gsl]) * ebi_v[gsl]
        acc = glob_vec + gb
        for d in range(D):
            cd = jnp.full((L,), d, jnp.int32)
            vu = plsc.load_gather(vmu_u, [rows, cd]) \
                + jnp.exp(0.5 * plsc.load_gather(vlv_u, [rows, cd])) \
                * plsc.load_gather(evu_v, [rows, cd])
            vi = plsc.load_gather(vmu_i, [rows, cd]) \
                + jnp.exp(0.5 * plsc.load_gather(vlv_i, [rows, cd])) \
                * plsc.load_gather(evi_v, [rows, cd])
            acc = acc + vu * vi
        out_v[pl.ds(g * L, L)] = acc
        return carry

    lax.fori_loop(0, NGRP, group, 0)
    pltpu.sync_copy(out_v, out_hbm.at[bsl])


@jax.jit
def kernel(u, i, user_bias_mu, user_bias_lv, user_vect_mu, user_vect_lv,
           item_bias_mu, item_bias_lv, item_vect_mu, item_vect_lv,
           glob_bias, eps_bu, eps_vu, eps_bi, eps_vi):
    mesh = plsc.VectorSubcoreMesh(core_axis_name="c", subcore_axis_name="s")
    f = pl.kernel(
        _body,
        mesh=mesh,
        compiler_params=pltpu.CompilerParams(
            needs_layout_passes=False, use_tc_tiling_on_sc=False),
        out_type=jax.ShapeDtypeStruct((B,), jnp.float32),
        scratch_types=[
            pltpu.VMEM((NCHUNK, 128), jnp.int32),
            pltpu.VMEM((NCHUNK, 128), jnp.int32),
            pltpu.VMEM((BW,), jnp.float32),
            pltpu.VMEM((BW,), jnp.float32),
            pltpu.VMEM((BW,), jnp.float32),
            pltpu.VMEM((BW,), jnp.float32),
            pltpu.VMEM((BW, D), jnp.float32),
            pltpu.VMEM((BW, D), jnp.float32),
            pltpu.VMEM((BW, D), jnp.float32),
            pltpu.VMEM((BW, D), jnp.float32),
            pltpu.VMEM((BW,), jnp.float32),
            pltpu.VMEM((BW,), jnp.float32),
            pltpu.VMEM((BW, D), jnp.float32),
            pltpu.VMEM((BW, D), jnp.float32),
            pltpu.VMEM((L,), jnp.float32),
            pltpu.VMEM((BW,), jnp.float32),
            pltpu.SemaphoreType.DMA,
        ],
    )
    return f(u, i,
             user_bias_mu.reshape(-1), user_bias_lv.reshape(-1),
             user_vect_mu, user_vect_lv,
             item_bias_mu.reshape(-1), item_bias_lv.reshape(-1),
             item_vect_mu, item_vect_lv,
             glob_bias.reshape(-1), eps_bu, eps_vu, eps_bi, eps_vi)


# speedup vs baseline: 3.5336x; 3.5336x over previous
import functools

import jax
import jax.numpy as jnp
from jax import lax
from jax.experimental import pallas as pl
from jax.experimental.pallas import tpu as pltpu
from jax.experimental.pallas import tpu_sc as plsc

B = 16384
D = 16
NU = 1000000

_info = plsc.get_sparse_core_info()
NC, NS, L = _info.num_cores, _info.num_subcores, _info.num_lanes
NW = NC * NS
BW = B // NW
NK = BW // 128
NGRP = BW // L

TC = (NU + 127) // 128
TROW = TC * 1024
FLAT = 2 * TROW
WIN = 64
GRID = (TC + WIN - 1) // WIN


def _detile_body(i0, i1, i2, i3, o0, o1, o2, o3):
    for i_ref, o_ref in ((i0, o0), (i1, o1), (i2, o2), (i3, o3)):
        for t in range(WIN):
            o_ref[0, t * 8:(t + 1) * 8, :] = i_ref[0:8, t * 128:(t + 1) * 128]
            o_ref[1, t * 8:(t + 1) * 8, :] = i_ref[8:16, t * 128:(t + 1) * 128]


def _detile(uvm, uvl, ivm, ivl):
    spec_in = pl.BlockSpec((D, WIN * 128), lambda j: (0, j))
    spec_out = pl.BlockSpec((2, WIN * 8, 128), lambda j: (0, j, 0))
    oshape = jax.ShapeDtypeStruct((2, TC * 8, 128), jnp.float32)
    return pl.pallas_call(
        _detile_body,
        grid=(GRID,),
        in_specs=[spec_in] * 4,
        out_specs=[spec_out] * 4,
        out_shape=[oshape] * 4,
    )(uvm, uvl, ivm, ivl)


def _sc_body(u_hbm, i_hbm,
             ubm_hbm, ubl_hbm, uvm_hbm, uvl_hbm,
             ibm_hbm, ibl_hbm, ivm_hbm, ivl_hbm,
             glob_hbm, ebu_hbm, evu_hbm, ebi_hbm, evi_hbm,
             out_hbm,
             u2d, i2d, idxs,
             bmu_u, blv_u, bmu_i, blv_i,
             vmu_u, vlv_u, vmu_i, vlv_i,
             ebu_v, ebi_v, evu_v, evi_v,
             glob_v, out_v, sem):
    wid = lax.axis_index("s") * NC + lax.axis_index("c")
    base = wid * BW

    for k in range(NK):
        pltpu.sync_copy(u_hbm.at[pl.ds(base + k * 128, 128)], u2d.at[k])
        pltpu.sync_copy(i_hbm.at[pl.ds(base + k * 128, 128)], i2d.at[k])
    pltpu.sync_copy(glob_hbm, glob_v.at[pl.ds(0, 1)])

    iota = lax.iota(jnp.int32, L)

    for side, src in ((0, u2d), (1, i2d)):
        for k in range(NK):
            for m in range(128 // L):
                x = src[k, pl.ds(m * L, L)]
                idxs[side, k, pl.ds(m * L, L)] = x + (x >> 7) * 896

    copies = []
    for tab, dst, side in ((uvm_hbm, vmu_u, 0), (uvl_hbm, vlv_u, 0),
                           (ivm_hbm, vmu_i, 1), (ivl_hbm, vlv_i, 1)):
        for d in range(D):
            cd = (d % 8) * 128 + (d // 8) * TROW
            win = tab.at[pl.ds(cd, FLAT - cd)]
            for k in range(NK):
                copies.append(pltpu.async_copy(
                    win.at[idxs.at[side, k]],
                    dst.at[d, pl.ds(k * 128, 128)], sem))
    for tab, dst, src in ((ubm_hbm, bmu_u, u2d), (ubl_hbm, blv_u, u2d),
                          (ibm_hbm, bmu_i, i2d), (ibl_hbm, blv_i, i2d)):
        for k in range(NK):
            copies.append(pltpu.async_copy(
                tab.at[src.at[k]],
                dst.at[pl.ds(k * 128, 128)], sem))
    bsl = pl.ds(base, BW)
    copies.append(pltpu.async_copy(evu_hbm.at[bsl, :], evu_v, sem))
    copies.append(pltpu.async_copy(evi_hbm.at[bsl, :], evi_v, sem))
    copies.append(pltpu.async_copy(ebu_hbm.at[bsl], ebu_v, sem))
    copies.append(pltpu.async_copy(ebi_hbm.at[bsl], ebi_v, sem))
    for c in copies:
        c.wait()

    zz = jnp.zeros((L,), jnp.int32)
    glob_vec = lax.gather(
        glob_v[...], zz.reshape(L, 1),
        dimension_numbers=lax.GatherDimensionNumbers(
            offset_dims=(), collapsed_slice_dims=(0,), start_index_map=(0,)),
        slice_sizes=(1,), mode=lax.GatherScatterMode.PROMISE_IN_BOUNDS)

    def group(g, carry):
        rows = g * L + iota
        gsl = pl.ds(g * L, L)
        gb = bmu_u[gsl] + jnp.exp(0.5 * blv_u[gsl]) * ebu_v[gsl] \
            + bmu_i[gsl] + jnp.exp(0.5 * blv_i[gsl]) * ebi_v[gsl]
        acc = glob_vec + gb
        for d in range(D):
            cd = jnp.full((L,), d, jnp.int32)
            vu = vmu_u[d, gsl] \
                + jnp.exp(0.5 * vlv_u[d, gsl]) \
                * plsc.load_gather(evu_v, [rows, cd])
            vi = vmu_i[d, gsl] \
                + jnp.exp(0.5 * vlv_i[d, gsl]) \
                * plsc.load_gather(evi_v, [rows, cd])
            acc = acc + vu * vi
        out_v[gsl] = acc
        return carry

    lax.fori_loop(0, NGRP, group, 0)
    pltpu.sync_copy(out_v, out_hbm.at[bsl])


@jax.jit
def kernel(u, i, user_bias_mu, user_bias_lv, user_vect_mu, user_vect_lv,
           item_bias_mu, item_bias_lv, item_vect_mu, item_vect_lv,
           glob_bias, eps_bu, eps_vu, eps_bi, eps_vi):
    uvm, uvl, ivm, ivl = _detile(user_vect_mu.T, user_vect_lv.T,
                                 item_vect_mu.T, item_vect_lv.T)
    mesh = plsc.VectorSubcoreMesh(core_axis_name="c", subcore_axis_name="s")
    f = pl.kernel(
        _sc_body,
        mesh=mesh,
        compiler_params=pltpu.CompilerParams(
            needs_layout_passes=False, use_tc_tiling_on_sc=False),
        out_type=jax.ShapeDtypeStruct((B,), jnp.float32),
        scratch_types=[
            pltpu.VMEM((NK, 128), jnp.int32),
            pltpu.VMEM((NK, 128), jnp.int32),
            pltpu.VMEM((2, NK, 128), jnp.int32),
            pltpu.VMEM((BW,), jnp.float32),
            pltpu.VMEM((BW,), jnp.float32),
            pltpu.VMEM((BW,), jnp.float32),
            pltpu.VMEM((BW,), jnp.float32),
            pltpu.VMEM((D, BW), jnp.float32),
            pltpu.VMEM((D, BW), jnp.float32),
            pltpu.VMEM((D, BW), jnp.float32),
            pltpu.VMEM((D, BW), jnp.float32),
            pltpu.VMEM((BW,), jnp.float32),
            pltpu.VMEM((BW,), jnp.float32),
            pltpu.VMEM((BW, D), jnp.float32),
            pltpu.VMEM((BW, D), jnp.float32),
            pltpu.VMEM((L,), jnp.float32),
            pltpu.VMEM((BW,), jnp.float32),
            pltpu.SemaphoreType.DMA,
        ],
    )
    return f(u, i,
             user_bias_mu.reshape(-1), user_bias_lv.reshape(-1),
             uvm.reshape(-1), uvl.reshape(-1),
             item_bias_mu.reshape(-1), item_bias_lv.reshape(-1),
             ivm.reshape(-1), ivl.reshape(-1),
             glob_bias.reshape(-1), eps_bu, eps_vu, eps_bi, eps_vi)


# speedup vs baseline: 3.6965x; 1.0461x over previous
import functools

import jax
import jax.numpy as jnp
from jax import lax
from jax.experimental import pallas as pl
from jax.experimental.pallas import tpu as pltpu
from jax.experimental.pallas import tpu_sc as plsc

B = 16384
D = 16
NU = 1000000

_info = plsc.get_sparse_core_info()
NC, NS, L = _info.num_cores, _info.num_subcores, _info.num_lanes
NW = NC * NS
BW = B // NW
NK = BW // 128
NGRP = BW // L

TC = (NU + 127) // 128
TROW = TC * 1024
FLAT = 2 * TROW
WIN = 256
GRID = (TC + WIN - 1) // WIN


def _detile_body(i0, i1, i2, i3, o0, o1, o2, o3):
    for i_ref, o_ref in ((i0, o0), (i1, o1), (i2, o2), (i3, o3)):
        for t in range(WIN):
            o_ref[0, t * 8:(t + 1) * 8, :] = i_ref[0:8, t * 128:(t + 1) * 128]
            o_ref[1, t * 8:(t + 1) * 8, :] = i_ref[8:16, t * 128:(t + 1) * 128]


def _detile(uvm, uvl, ivm, ivl):
    spec_in = pl.BlockSpec((D, WIN * 128), lambda j: (0, j))
    spec_out = pl.BlockSpec((2, WIN * 8, 128), lambda j: (0, j, 0))
    oshape = jax.ShapeDtypeStruct((2, TC * 8, 128), jnp.float32)
    return pl.pallas_call(
        _detile_body,
        grid=(GRID,),
        in_specs=[spec_in] * 4,
        out_specs=[spec_out] * 4,
        out_shape=[oshape] * 4,
    )(uvm, uvl, ivm, ivl)


def _sc_body(u_hbm, i_hbm,
             ubm_hbm, ubl_hbm, uvm_hbm, uvl_hbm,
             ibm_hbm, ibl_hbm, ivm_hbm, ivl_hbm,
             glob_hbm, ebu_hbm, evu_hbm, ebi_hbm, evi_hbm,
             out_hbm,
             u2d, i2d, idxs,
             bmu_u, blv_u, bmu_i, blv_i,
             vmu_u, vlv_u, vmu_i, vlv_i,
             ebu_v, ebi_v, evu_v, evi_v,
             glob_v, out_v, sem):
    wid = lax.axis_index("s") * NC + lax.axis_index("c")
    base = wid * BW

    for k in range(NK):
        pltpu.sync_copy(u_hbm.at[pl.ds(base + k * 128, 128)], u2d.at[k])
        pltpu.sync_copy(i_hbm.at[pl.ds(base + k * 128, 128)], i2d.at[k])
    pltpu.sync_copy(glob_hbm, glob_v.at[pl.ds(0, 1)])

    iota = lax.iota(jnp.int32, L)

    for side, src in ((0, u2d), (1, i2d)):
        for k in range(NK):
            for m in range(128 // L):
                x = src[k, pl.ds(m * L, L)]
                idxs[side, k, pl.ds(m * L, L)] = x + (x >> 7) * 896

    copies = []
    for tab, dst, side in ((uvm_hbm, vmu_u, 0), (uvl_hbm, vlv_u, 0),
                           (ivm_hbm, vmu_i, 1), (ivl_hbm, vlv_i, 1)):
        for d in range(D):
            cd = (d % 8) * 128 + (d // 8) * TROW
            win = tab.at[pl.ds(cd, FLAT - cd)]
            for k in range(NK):
                copies.append(pltpu.async_copy(
                    win.at[idxs.at[side, k]],
                    dst.at[d, pl.ds(k * 128, 128)], sem))
    for tab, dst, src in ((ubm_hbm, bmu_u, u2d), (ubl_hbm, blv_u, u2d),
                          (ibm_hbm, bmu_i, i2d), (ibl_hbm, blv_i, i2d)):
        for k in range(NK):
            copies.append(pltpu.async_copy(
                tab.at[src.at[k]],
                dst.at[pl.ds(k * 128, 128)], sem))
    bsl = pl.ds(base, BW)
    copies.append(pltpu.async_copy(evu_hbm.at[bsl, :], evu_v, sem))
    copies.append(pltpu.async_copy(evi_hbm.at[bsl, :], evi_v, sem))
    copies.append(pltpu.async_copy(ebu_hbm.at[bsl], ebu_v, sem))
    copies.append(pltpu.async_copy(ebi_hbm.at[bsl], ebi_v, sem))
    for c in copies:
        c.wait()

    zz = jnp.zeros((L,), jnp.int32)
    glob_vec = lax.gather(
        glob_v[...], zz.reshape(L, 1),
        dimension_numbers=lax.GatherDimensionNumbers(
            offset_dims=(), collapsed_slice_dims=(0,), start_index_map=(0,)),
        slice_sizes=(1,), mode=lax.GatherScatterMode.PROMISE_IN_BOUNDS)

    def group(g, carry):
        rows = g * L + iota
        gsl = pl.ds(g * L, L)
        gb = bmu_u[gsl] + jnp.exp(0.5 * blv_u[gsl]) * ebu_v[gsl] \
            + bmu_i[gsl] + jnp.exp(0.5 * blv_i[gsl]) * ebi_v[gsl]
        acc = glob_vec + gb
        for d in range(D):
            cd = jnp.full((L,), d, jnp.int32)
            vu = vmu_u[d, gsl] \
                + jnp.exp(0.5 * vlv_u[d, gsl]) \
                * plsc.load_gather(evu_v, [rows, cd])
            vi = vmu_i[d, gsl] \
                + jnp.exp(0.5 * vlv_i[d, gsl]) \
                * plsc.load_gather(evi_v, [rows, cd])
            acc = acc + vu * vi
        out_v[gsl] = acc
        return carry

    lax.fori_loop(0, NGRP, group, 0)
    pltpu.sync_copy(out_v, out_hbm.at[bsl])


@jax.jit
def kernel(u, i, user_bias_mu, user_bias_lv, user_vect_mu, user_vect_lv,
           item_bias_mu, item_bias_lv, item_vect_mu, item_vect_lv,
           glob_bias, eps_bu, eps_vu, eps_bi, eps_vi):
    uvm, uvl, ivm, ivl = _detile(user_vect_mu.T, user_vect_lv.T,
                                 item_vect_mu.T, item_vect_lv.T)
    mesh = plsc.VectorSubcoreMesh(core_axis_name="c", subcore_axis_name="s")
    f = pl.kernel(
        _sc_body,
        mesh=mesh,
        compiler_params=pltpu.CompilerParams(
            needs_layout_passes=False, use_tc_tiling_on_sc=False),
        out_type=jax.ShapeDtypeStruct((B,), jnp.float32),
        scratch_types=[
            pltpu.VMEM((NK, 128), jnp.int32),
            pltpu.VMEM((NK, 128), jnp.int32),
            pltpu.VMEM((2, NK, 128), jnp.int32),
            pltpu.VMEM((BW,), jnp.float32),
            pltpu.VMEM((BW,), jnp.float32),
            pltpu.VMEM((BW,), jnp.float32),
            pltpu.VMEM((BW,), jnp.float32),
            pltpu.VMEM((D, BW), jnp.float32),
            pltpu.VMEM((D, BW), jnp.float32),
            pltpu.VMEM((D, BW), jnp.float32),
            pltpu.VMEM((D, BW), jnp.float32),
            pltpu.VMEM((BW,), jnp.float32),
            pltpu.VMEM((BW,), jnp.float32),
            pltpu.VMEM((BW, D), jnp.float32),
            pltpu.VMEM((BW, D), jnp.float32),
            pltpu.VMEM((L,), jnp.float32),
            pltpu.VMEM((BW,), jnp.float32),
            pltpu.SemaphoreType.DMA,
        ],
    )
    return f(u, i,
             user_bias_mu.reshape(-1), user_bias_lv.reshape(-1),
             uvm.reshape(-1), uvl.reshape(-1),
             item_bias_mu.reshape(-1), item_bias_lv.reshape(-1),
             ivm.reshape(-1), ivl.reshape(-1),
             glob_bias.reshape(-1), eps_bu, eps_vu, eps_bi, eps_vi)
